# baseline (speedup 1.0000x reference)
import functools

import jax
import jax.numpy as jnp
from jax import lax
from jax.experimental import pallas as pl
from jax.experimental.pallas import tpu as pltpu
from jax.experimental.pallas import tpu_sc as plsc

NC = 2
NS = 16
CH = 64
DEGW = 16

_MESH = plsc.VectorSubcoreMesh(
    core_axis_name="c", subcore_axis_name="s", num_cores=NC, num_subcores=NS)


def _gather_body(ept_w, h_hbm, idx_hbm, hg_hbm, idx_v0, idx_v1, gbuf0, gbuf1,
                 sem0, sem1):
  c = lax.axis_index("c")
  s = lax.axis_index("s")
  wid = s * NC + c
  base = wid * ept_w

  def chunk(k, _):
    e0 = base + 2 * k * CH
    e1 = e0 + CH
    pltpu.sync_copy(idx_hbm.at[pl.ds(e0, CH)], idx_v0)
    pltpu.sync_copy(idx_hbm.at[pl.ds(e1, CH)], idx_v1)
    d0 = pltpu.async_copy(h_hbm.at[idx_v0], gbuf0, sem0)
    d1 = pltpu.async_copy(h_hbm.at[idx_v1], gbuf1, sem1)
    d0.wait()
    pltpu.sync_copy(gbuf0, hg_hbm.at[pl.ds(e0, CH), :])
    d1.wait()
    pltpu.sync_copy(gbuf1, hg_hbm.at[pl.ds(e1, CH), :])
    return 0
  lax.fori_loop(0, ept_w // (2 * CH), chunk, 0)


def _prelude_body(x_ref, m_ref, g_ref, bt_ref, w_ref, deg_ref, h_ref):
  xb = x_ref[...]
  mu = jnp.mean(xb, axis=1, keepdims=True)
  xc = xb - mu
  var = jnp.mean(xc * xc, axis=1, keepdims=True)
  y = xc * lax.rsqrt(var + 1e-5) * g_ref[...] + bt_ref[...]
  y = jnp.maximum(y, 0.0) * m_ref[...]
  h = jnp.dot(y, w_ref[...], preferred_element_type=jnp.float32)
  dinv = lax.rsqrt(deg_ref[...][:, 0:1] + 1.0)
  h_ref[...] = h * dinv


def _epi_body(a_ref, h_ref, deg_ref, b_ref, o_ref):
  dinv = lax.rsqrt(deg_ref[...][:, 0:1] + 1.0)
  o_ref[...] = dinv * (a_ref[...] + h_ref[...]) + b_ref[...]


def kernel(x, edge_index, dropout_mask, gamma, beta, W, b):
  n, d = x.shape
  e = edge_index.shape[1]

  src = edge_index[0].astype(jnp.int32)
  dst = edge_index[1].astype(jnp.int32)

  e_pad = -(-e // (NC * NS * 2 * CH)) * (NC * NS * 2 * CH)
  src_p = jnp.concatenate([src, jnp.zeros((e_pad - e,), jnp.int32)])

  degv = jax.ops.segment_sum(jnp.ones((e,), jnp.float32), dst,
                             num_segments=n)
  deg16 = jnp.broadcast_to(degv[:, None], (n, DEGW))

  rb = 400
  grid = n // rb
  g2 = gamma.reshape(1, d)
  bt2 = beta.reshape(1, d)
  hs = pl.pallas_call(
      _prelude_body,
      grid=(grid,),
      in_specs=[
          pl.BlockSpec((rb, d), lambda i: (i, 0)),
          pl.BlockSpec((rb, d), lambda i: (i, 0)),
          pl.BlockSpec((1, d), lambda i: (0, 0)),
          pl.BlockSpec((1, d), lambda i: (0, 0)),
          pl.BlockSpec((d, d), lambda i: (0, 0)),
          pl.BlockSpec((rb, DEGW), lambda i: (i, 0)),
      ],
      out_specs=pl.BlockSpec((rb, d), lambda i: (i, 0)),
      out_shape=jax.ShapeDtypeStruct((n, d), jnp.float32),
  )(x, dropout_mask, g2, bt2, W, deg16)

  ept_w = e_pad // (NC * NS)
  gather_kernel = pl.kernel(
      functools.partial(_gather_body, ept_w),
      out_type=jax.ShapeDtypeStruct((e_pad, d), jnp.float32),
      mesh=_MESH,
      scratch_types=[
          pltpu.VMEM((CH,), jnp.int32),
          pltpu.VMEM((CH,), jnp.int32),
          pltpu.VMEM((CH, d), jnp.float32),
          pltpu.VMEM((CH, d), jnp.float32),
          pltpu.SemaphoreType.DMA,
          pltpu.SemaphoreType.DMA,
      ],
  )
  hg = gather_kernel(hs, src_p)

  agg = jax.ops.segment_sum(hg[:e], dst, num_segments=n)

  b2 = b.reshape(1, d)
  out = pl.pallas_call(
      _epi_body,
      grid=(grid,),
      in_specs=[
          pl.BlockSpec((rb, d), lambda i: (i, 0)),
          pl.BlockSpec((rb, d), lambda i: (i, 0)),
          pl.BlockSpec((rb, DEGW), lambda i: (i, 0)),
          pl.BlockSpec((1, d), lambda i: (0, 0)),
      ],
      out_specs=pl.BlockSpec((rb, d), lambda i: (i, 0)),
      out_shape=jax.ShapeDtypeStruct((n, d), jnp.float32),
  )(agg, hs, deg16, b2)
  return out
